# Initial kernel scaffold; baseline (speedup 1.0000x reference)
import jax
import jax.numpy as jnp
from jax.experimental import pallas as pl


def kernel(x, w1, b1, w2, b2):
    raise NotImplementedError("write your pallas kernel here")


# speedup vs baseline: 1.4706x; 1.4706x over previous
import math

import jax
import jax.numpy as jnp
from jax.experimental import pallas as pl
from jax.experimental.pallas import tpu as pltpu

_INV_SQRT2 = 1.0 / math.sqrt(2.0)


def _round_up(a, b):
    return (a + b - 1) // b * b


def _make_body(nh_chunks, th):
    def _body(x_ref, w1_ref, b1_ref, w2_ref, b2_ref, o_ref):
        xb = x_ref[...].astype(jnp.bfloat16)
        acc = b2_ref[...].astype(jnp.float32)
        for j in range(nh_chunks):
            sl = slice(j * th, (j + 1) * th)
            h = jnp.dot(xb, w1_ref[:, sl].astype(jnp.bfloat16),
                        preferred_element_type=jnp.float32)
            h = h + b1_ref[0, sl].astype(jnp.float32)
            g = 0.5 * h * (1.0 + jax.lax.erf(h * _INV_SQRT2))
            acc = acc + jnp.dot(g.astype(jnp.bfloat16),
                                w2_ref[sl, :].astype(jnp.bfloat16),
                                preferred_element_type=jnp.float32)
        o_ref[...] = acc.astype(o_ref.dtype)

    return _body


def kernel(x, w1, b1, w2, b2):
    B, S, dim = x.shape
    hidden = w1.shape[1]
    M = B * S

    dim_p = _round_up(dim, 128)
    tm = 512 if M >= 512 else _round_up(M, 8)
    M_p = _round_up(M, tm)
    th = 512 if hidden >= 512 else _round_up(hidden, 128)
    hidden_p = _round_up(hidden, th)
    nh = hidden_p // th

    x2d = jnp.pad(x.reshape(M, dim), ((0, M_p - M), (0, dim_p - dim)))
    w1p = jnp.pad(w1, ((0, dim_p - dim), (0, hidden_p - hidden)))
    b1p = jnp.pad(b1, (0, hidden_p - hidden)).reshape(1, hidden_p)
    w2p = jnp.pad(w2, ((0, hidden_p - hidden), (0, dim_p - dim)))
    b2p = jnp.pad(b2, (0, dim_p - dim)).reshape(1, dim_p)

    grid = (M_p // tm,)

    out2d = pl.pallas_call(
        _make_body(nh, th),
        out_shape=jax.ShapeDtypeStruct((M_p, dim_p), x.dtype),
        grid=grid,
        in_specs=[
            pl.BlockSpec((tm, dim_p), lambda i: (i, 0)),
            pl.BlockSpec((dim_p, hidden_p), lambda i: (0, 0)),
            pl.BlockSpec((1, hidden_p), lambda i: (0, 0)),
            pl.BlockSpec((hidden_p, dim_p), lambda i: (0, 0)),
            pl.BlockSpec((1, dim_p), lambda i: (0, 0)),
        ],
        out_specs=pl.BlockSpec((tm, dim_p), lambda i: (i, 0)),
        compiler_params=pltpu.CompilerParams(
            dimension_semantics=("parallel",),
            vmem_limit_bytes=100 * 1024 * 1024,
        ),
    )(x2d, w1p, b1p, w2p, b2p)

    return out2d[:M, :dim].reshape(B, S, dim)
